# baseline (speedup 1.0000x reference)
import functools
import jax
import jax.numpy as jnp
from jax import lax
from jax.experimental import pallas as pl
from jax.experimental.pallas import tpu as pltpu
from jax.experimental.pallas import tpu_sc as plsc

N = 10000
D = 128
D2 = D // 2
H = 64
E = 160000
T = 6 * E
NC = 2
NS = 16
L = 16
CHUNK = 80
BLKE = 2000
SUBS = BLKE // CHUNK
LBLK = 400
LCH = 40
LSUBS = LBLK // LCH
EPT = E // NS
RPT = 640
RPT_LAST = N - 15 * RPT


def _sc_mesh():
  return plsc.VectorSubcoreMesh(
      core_axis_name="c", subcore_axis_name="s", num_cores=NC,
      num_subcores=NS)


@functools.cache
def _make_agg(with_deg):
  out_type = [jax.ShapeDtypeStruct((2, NC, N, D2), jnp.float32)]
  scratch = [
      pltpu.VMEM((BLKE,), jnp.int32),
      pltpu.VMEM((SUBS, CHUNK), jnp.int32),
      pltpu.VMEM((CHUNK, D2), jnp.float32),
      pltpu.VMEM((CHUNK, D2), jnp.float32),
      pltpu.VMEM((RPT, D2), jnp.float32),
      pltpu.VMEM_SHARED((N, D2), jnp.float32),
      pltpu.SemaphoreType.DMA,
  ]
  if with_deg:
    out_type.append(jax.ShapeDtypeStruct((2, N, L), jnp.float32))
    scratch += [
        pltpu.VMEM((CHUNK, L), jnp.float32),
        pltpu.VMEM((RPT, L), jnp.float32),
        pltpu.VMEM_SHARED((N, L), jnp.float32),
    ]

  @functools.partial(
      pl.kernel, out_type=tuple(out_type), mesh=_sc_mesh(),
      scratch_types=tuple(scratch),
      compiler_params=pltpu.CompilerParams(use_tc_tiling_on_sc=False))
  def agg_kernel(src_cat, dst2d, table_cat, z_rows, *refs):
    if with_deg:
      (z_rows16, ones16, agg_out, deg_out, idx_s, idx_d2, rows0, rows1, wb,
       acc_sh, sem, ones_v, wbd, deg_sh) = refs
    else:
      (agg_out, idx_s, idx_d2, rows0, rows1, wb, acc_sh, sem) = refs
    rowbufs = (rows0, rows1)
    cid = lax.axis_index("c")
    sid = lax.axis_index("s")
    row0 = sid * RPT
    cofs = cid * N

    if with_deg:
      pltpu.sync_copy(ones16, ones_v)

    for p in range(2):
      @pl.when(sid < NS - 1)
      def _():
        pltpu.sync_copy(z_rows, acc_sh.at[pl.ds(row0, RPT)])
        if with_deg:
          @pl.when(cid == p)
          def _():
            pltpu.sync_copy(z_rows16, deg_sh.at[pl.ds(row0, RPT)])

      @pl.when(sid == NS - 1)
      def _():
        pltpu.sync_copy(z_rows.at[pl.ds(0, RPT_LAST)],
                        acc_sh.at[pl.ds(row0, RPT_LAST)])
        if with_deg:
          @pl.when(cid == p)
          def _():
            pltpu.sync_copy(z_rows16.at[pl.ds(0, RPT_LAST)],
                            deg_sh.at[pl.ds(row0, RPT_LAST)])

      plsc.subcore_barrier()

      ebase = p * E + sid * EPT

      erow = p * (E // CHUNK) + sid * (EPT // CHUNK)

      def block_body(b, carry):
        boff = ebase + b * BLKE
        brow = erow + b * SUBS
        pltpu.sync_copy(src_cat.at[pl.ds(boff, BLKE)], idx_s)
        pltpu.sync_copy(dst2d.at[pl.ds(brow, SUBS)], idx_d2)

        def shift(k, c):
          sl = pl.ds(k * L, L)
          idx_s[sl] = idx_s[sl] + cofs
          return c

        lax.fori_loop(0, BLKE // L, shift, 0)

        descs = [None, None]
        descs[0] = pltpu.async_copy(
            table_cat.at[idx_s.at[pl.ds(0, CHUNK)]], rowbufs[0], sem)
        for k in range(SUBS):
          if k + 1 < SUBS:
            descs[(k + 1) % 2] = pltpu.async_copy(
                table_cat.at[idx_s.at[pl.ds((k + 1) * CHUNK, CHUNK)]],
                rowbufs[(k + 1) % 2], sem)
          descs[k % 2].wait()
          pltpu.sync_copy(rowbufs[k % 2], acc_sh.at[idx_d2.at[k]],
                          add=True)
          if with_deg:
            @pl.when(cid == p)
            def _():
              pltpu.sync_copy(ones_v, deg_sh.at[idx_d2.at[k]], add=True)
        return carry

      lax.fori_loop(0, EPT // BLKE, block_body, 0)
      plsc.subcore_barrier()

      @pl.when(sid < NS - 1)
      def _():
        pltpu.sync_copy(acc_sh.at[pl.ds(row0, RPT)], wb)
        pltpu.sync_copy(wb, agg_out.at[p, cid, pl.ds(row0, RPT)])
        if with_deg:
          @pl.when(cid == p)
          def _():
            pltpu.sync_copy(deg_sh.at[pl.ds(row0, RPT)], wbd)
            pltpu.sync_copy(wbd, deg_out.at[p, pl.ds(row0, RPT)])

      @pl.when(sid == NS - 1)
      def _():
        pltpu.sync_copy(acc_sh.at[pl.ds(row0, RPT_LAST)],
                        wb.at[pl.ds(0, RPT_LAST)])
        pltpu.sync_copy(wb.at[pl.ds(0, RPT_LAST)],
                        agg_out.at[p, cid, pl.ds(row0, RPT_LAST)])
        if with_deg:
          @pl.when(cid == p)
          def _():
            pltpu.sync_copy(deg_sh.at[pl.ds(row0, RPT_LAST)],
                            wbd.at[pl.ds(0, RPT_LAST)])
            pltpu.sync_copy(wbd.at[pl.ds(0, RPT_LAST)],
                            deg_out.at[p, pl.ds(row0, RPT_LAST)])

  return agg_kernel


@functools.cache
def _make_loss():
  @functools.partial(
      pl.kernel,
      out_type=(
          jax.ShapeDtypeStruct((T,), jnp.float32),
          jax.ShapeDtypeStruct((T,), jnp.float32),
          jax.ShapeDtypeStruct((T,), jnp.float32),
          jax.ShapeDtypeStruct((NC, NS, 1, L), jnp.float32),
      ),
      mesh=_sc_mesh(),
      compiler_params=pltpu.CompilerParams(
          needs_layout_passes=False, use_tc_tiling_on_sc=False),
      scratch_types=(
          pltpu.VMEM((N, 6), jnp.float32),
          pltpu.VMEM((LBLK,), jnp.int32),
          pltpu.VMEM((LBLK,), jnp.int32),
          pltpu.VMEM((LBLK,), jnp.int32),
          pltpu.VMEM((LCH, D), jnp.float32),
          pltpu.VMEM((LCH, D), jnp.float32),
          pltpu.VMEM((LCH, D), jnp.float32),
          pltpu.VMEM((LCH, D), jnp.float32),
          pltpu.VMEM((LCH, D), jnp.float32),
          pltpu.VMEM((LCH, D), jnp.float32),
          pltpu.VMEM((9, LBLK), jnp.float32),
          pltpu.VMEM((1, L), jnp.float32),
          pltpu.SemaphoreType.DMA,
      ),
  )
  def loss_kernel(z_hbm, pt_hbm, src_cat, dst_cat, surr_cat,
                  l0_out, l1_out, l2_out, hinge_out,
                  pt_v, idx_i, idx_j, idx_k, zi0, zj0, zk0, zi1, zj1, zk1,
                  lbuf, hbuf, sem):
    cid = lax.axis_index("c")
    sid = lax.axis_index("s")
    bufs = ((zi0, zj0, zk0), (zi1, zj1, zk1))
    pltpu.sync_copy(pt_hbm, pt_v)

    goff0 = cid * E
    goff1 = (4 - 2 * cid) * E
    goff2 = (5 - 2 * cid) * E
    sgn = (1.0 - 2.0 * cid.astype(jnp.float32))

    base = sid * EPT

    def fire(k, which):
      sl = pl.ds(k * LCH, LCH)
      zi, zj, zk = bufs[which]
      return (pltpu.async_copy(z_hbm.at[idx_i.at[sl]], zi, sem),
              pltpu.async_copy(z_hbm.at[idx_j.at[sl]], zj, sem),
              pltpu.async_copy(z_hbm.at[idx_k.at[sl]], zk, sem))

    def block_body(b, hacc):
      boff = base + b * LBLK
      eoff = cid * E + boff
      pltpu.sync_copy(src_cat.at[pl.ds(eoff, LBLK)], idx_i)
      pltpu.sync_copy(dst_cat.at[pl.ds(eoff, LBLK)], idx_j)
      pltpu.sync_copy(surr_cat.at[pl.ds(eoff, LBLK)], idx_k)

      descs = [None, None]
      descs[0] = fire(0, 0)
      for k in range(LSUBS):
        if k + 1 < LSUBS:
          descs[(k + 1) % 2] = fire(k + 1, (k + 1) % 2)
        for d in descs[k % 2]:
          d.wait()
        zi, zj, zk = bufs[k % 2]

        def row_body(r, acc, zi=zi, zj=zj, zk=zk):
          aij = jnp.zeros((L,), jnp.float32)
          aik = jnp.zeros((L,), jnp.float32)
          for c in range(D // L):
            vi = zi[r, pl.ds(c * L, L)]
            vj = zj[r, pl.ds(c * L, L)]
            vk = zk[r, pl.ds(c * L, L)]
            dij = vi - vj
            dik = vi - vk
            aij = aij + dij * dij
            aik = aik + dik * dik
          return acc + jnp.maximum(sgn * jnp.sum(aij - aik), 0.0)

        hacc = lax.fori_loop(0, LCH, row_body, hacc)

      for g in range(LBLK // L):
        gsl = pl.ds(g * L, L)
        iv = idx_i[gsl]
        jv = idx_j[gsl]
        kv = idx_k[gsl]
        for p, (av, bv) in enumerate(((iv, jv), (iv, kv), (jv, kv))):
          for c in range(3):
            la = plsc.load_gather(
                pt_v, [av, jnp.full((L,), c, jnp.int32)])
            lb = plsc.load_gather(
                pt_v, [bv, jnp.full((L,), c + 3, jnp.int32)])
            lbuf[3 * p + c, gsl] = la + lb

      for p, goff in enumerate((goff0, goff1, goff2)):
        for c, out in enumerate((l0_out, l1_out, l2_out)):
          pltpu.sync_copy(lbuf.at[3 * p + c],
                          out.at[pl.ds(goff + boff, LBLK)])
      return hacc

    hsum = lax.fori_loop(0, EPT // LBLK, block_body, 0.0)
    hbuf[0, :] = jnp.full((L,), hsum, jnp.float32)
    pltpu.sync_copy(hbuf, hinge_out.at[cid, sid])

  return loss_kernel


_BLK = 1000


def _halves(ref_lo, ref_hi):
  return jnp.concatenate([ref_lo[0, 0], ref_hi[0, 0]], axis=1)


def _sage_core(x, apl, aph, anl, anh, degp_ref, degn_ref,
               wp_ref, bp_ref, wn_ref, bn_ref):
  invp = 1.0 / (degp_ref[...][0, :, :1] + 1.0)
  invn = 1.0 / (degn_ref[...][0, :, :1] + 1.0)
  meanp = (_halves(apl, aph) + x) * invp
  meann = (_halves(anl, anh) + x) * invn
  hp = jnp.tanh(
      jnp.dot(meanp, wp_ref[...], preferred_element_type=jnp.float32)
      + bp_ref[...])
  hn = jnp.tanh(
      jnp.dot(meann, wn_ref[...], preferred_element_type=jnp.float32)
      + bn_ref[...])
  return hp, hn


def _blkmaps(n_agg):
  specs = []
  for s in range(2):
    for h in range(2):
      specs.append(pl.BlockSpec(
          (1, 1, _BLK, D2),
          functools.partial(lambda s, h, i: (s, h, i, 0), s, h)))
  return specs


def _sage1_body(x_ref, apl, aph, anl, anh, degp_ref, degn_ref,
                wp_ref, bp_ref, wn_ref, bn_ref, out_ref):
  hp, hn = _sage_core(x_ref[...], apl, aph, anl, anh, degp_ref, degn_ref,
                      wp_ref, bp_ref, wn_ref, bn_ref)
  out_ref[0] = hp
  out_ref[1] = hn


def _sage_layer1(x, agg4, deg2, wp, bp, wn, bn):
  full = lambda i: (0, 0)
  return pl.pallas_call(
      _sage1_body,
      grid=(N // _BLK,),
      in_specs=[
          pl.BlockSpec((_BLK, D), lambda i: (i, 0)),
          *_blkmaps(agg4),
          pl.BlockSpec((1, _BLK, L), lambda i: (0, i, 0)),
          pl.BlockSpec((1, _BLK, L), lambda i: (1, i, 0)),
          pl.BlockSpec((D, H), full),
          pl.BlockSpec((1, H), full),
          pl.BlockSpec((D, H), full),
          pl.BlockSpec((1, H), full),
      ],
      out_specs=pl.BlockSpec((2, _BLK, H), lambda i: (0, i, 0)),
      out_shape=jax.ShapeDtypeStruct((2, N, H), jnp.float32),
  )(x, agg4, agg4, agg4, agg4, deg2, deg2, wp, bp, wn, bn)


def _sage2_body(cp_ref, cn_ref, apl, aph, anl, anh, degp_ref, degn_ref,
                wp_ref, bp_ref, wn_ref, bn_ref, w1_ref, w2_ref,
                z_ref, pt_ref):
  x = jnp.concatenate([cp_ref[0], cn_ref[0]], axis=1)
  hp, hn = _sage_core(x, apl, aph, anl, anh, degp_ref, degn_ref,
                      wp_ref, bp_ref, wn_ref, bn_ref)
  z = jnp.concatenate([hp, hn], axis=1)
  z_ref[...] = z
  p1 = jnp.dot(z, w1_ref[...], preferred_element_type=jnp.float32)
  p2 = jnp.dot(z, w2_ref[...], preferred_element_type=jnp.float32)
  pt_ref[...] = jnp.concatenate([p1, p2], axis=1)


def _sage_layer2(c2, agg4, deg2, wp, bp, wn, bn, w1, w2):
  full = lambda i: (0, 0)
  return pl.pallas_call(
      _sage2_body,
      grid=(N // _BLK,),
      in_specs=[
          pl.BlockSpec((1, _BLK, H), lambda i: (0, i, 0)),
          pl.BlockSpec((1, _BLK, H), lambda i: (1, i, 0)),
          *_blkmaps(agg4),
          pl.BlockSpec((1, _BLK, L), lambda i: (0, i, 0)),
          pl.BlockSpec((1, _BLK, L), lambda i: (1, i, 0)),
          pl.BlockSpec((D, H), full),
          pl.BlockSpec((1, H), full),
          pl.BlockSpec((D, H), full),
          pl.BlockSpec((1, H), full),
          pl.BlockSpec((D, 3), full),
          pl.BlockSpec((D, 3), full),
      ],
      out_specs=[
          pl.BlockSpec((_BLK, D), lambda i: (i, 0)),
          pl.BlockSpec((_BLK, 6), lambda i: (i, 0)),
      ],
      out_shape=[
          jax.ShapeDtypeStruct((N, D), jnp.float32),
          jax.ShapeDtypeStruct((N, 6), jnp.float32),
      ],
  )(c2, c2, agg4, agg4, agg4, agg4, deg2, deg2, wp, bp, wn, bn, w1, w2)


_TROWS = T // D


def _final_body(l0_ref, l1_ref, l2_ref, tgt_ref, hinge_ref,
                wp0_ref, wn0_ref, wp1_ref, wn1_ref, rw_ref, out_ref):
  i = pl.program_id(0)
  l0 = l0_ref[...]
  l1 = l1_ref[...]
  l2 = l2_ref[...]
  t = tgt_ref[...]
  m = jnp.maximum(jnp.maximum(l0, l1), l2)
  lse = m + jnp.log(jnp.exp(l0 - m) + jnp.exp(l1 - m) + jnp.exp(l2 - m))
  lt = jnp.where(t == 0, l0, jnp.where(t == 1, l1, l2))
  part = jnp.sum(lse - lt)

  @pl.when(i == 0)
  def _():
    out_ref[...] = jnp.zeros((1, 1), jnp.float32)

  out_ref[...] += jnp.full((1, 1), part / T, jnp.float32)

  @pl.when(i == pl.num_programs(0) - 1)
  def _():
    hp = jnp.sum(hinge_ref[0, :, 0]) / E
    hn = jnp.sum(hinge_ref[1, :, 0]) / E

    def rnm(w):
      return jnp.mean(jnp.sqrt(jnp.sum(w * w, axis=1)))

    reg = (rnm(wp0_ref[...]) + rnm(wn0_ref[...]) + rnm(rw_ref[...])
           + rnm(wp1_ref[...]) + rnm(wn1_ref[...]))
    out_ref[...] += jnp.full((1, 1), hp + hn + 0.01 * reg, jnp.float32)


def _final_loss(l0, l1, l2, tgt, hinge, wp0, wn0, wp1, wn1, rw):
  full = lambda i: (0, 0)
  full3 = lambda i: (0, 0, 0)
  return pl.pallas_call(
      _final_body,
      grid=(1,),
      in_specs=[
          pl.BlockSpec((_TROWS, D), lambda i: (i, 0)),
          pl.BlockSpec((_TROWS, D), lambda i: (i, 0)),
          pl.BlockSpec((_TROWS, D), lambda i: (i, 0)),
          pl.BlockSpec((_TROWS, D), lambda i: (i, 0)),
          pl.BlockSpec((NC, NS, L), full3),
          pl.BlockSpec((D, H), full),
          pl.BlockSpec((D, H), full),
          pl.BlockSpec((D, H), full),
          pl.BlockSpec((D, H), full),
          pl.BlockSpec((2 * D, 3), full),
      ],
      out_specs=pl.BlockSpec((1, 1), full),
      out_shape=jax.ShapeDtypeStruct((1, 1), jnp.float32),
  )(l0, l1, l2, tgt, hinge, wp0, wn0, wp1, wn1, rw)


def kernel(positive_edges, negative_edges, target, pos_surr, neg_surr, X,
           W_pos0, b_pos0, W_neg0, b_neg0, W_pos1, b_pos1, W_neg1, b_neg1,
           reg_W):
  src_cat = jnp.concatenate([positive_edges[0], negative_edges[0]])
  dst_cat = jnp.concatenate([positive_edges[1], negative_edges[1]])
  surr_cat = jnp.concatenate([pos_surr, neg_surr])
  x_cat = jnp.concatenate([X[:, :D2], X[:, D2:]], axis=0)
  z_rows = jnp.zeros((RPT, D2), jnp.float32)
  z_rows16 = jnp.zeros((RPT, L), jnp.float32)
  ones16 = jnp.ones((CHUNK, L), jnp.float32)

  dst2d = dst_cat.reshape(-1, CHUNK)
  agg4a, deg2 = _make_agg(True)(src_cat, dst2d, x_cat, z_rows, z_rows16,
                                ones16)
  c2 = _sage_layer1(X, agg4a, deg2,
                    W_pos0, b_pos0[None, :], W_neg0, b_neg0[None, :])
  agg4b = _make_agg(False)(src_cat, dst2d, c2.reshape(2 * N, H), z_rows)
  if isinstance(agg4b, (tuple, list)):
    agg4b = agg4b[0]
  w_neg1_sw = jnp.concatenate([W_neg1[H:], W_neg1[:H]], axis=0)
  z, pt = _sage_layer2(c2, agg4b, deg2,
                       W_pos1, b_pos1[None, :], w_neg1_sw, b_neg1[None, :],
                       reg_W[:D], reg_W[D:])
  l0, l1, l2, hinge = _make_loss()(z, pt, src_cat, dst_cat, surr_cat)
  hinge = hinge.reshape(NC, NS, L)
  loss = _final_loss(l0.reshape(_TROWS, D), l1.reshape(_TROWS, D),
                     l2.reshape(_TROWS, D), target.reshape(_TROWS, D),
                     hinge, W_pos0, W_neg0, W_pos1, W_neg1, reg_W)
  return (loss[0, 0], z)


# speedup vs baseline: 1.0386x; 1.0386x over previous
import functools
import jax
import jax.numpy as jnp
from jax import lax
from jax.experimental import pallas as pl
from jax.experimental.pallas import tpu as pltpu
from jax.experimental.pallas import tpu_sc as plsc

N = 10000
D = 128
D2 = D // 2
H = 64
E = 160000
T = 6 * E
NC = 2
NS = 16
L = 16
CHUNK = 80
BLKE = 2000
SUBS = BLKE // CHUNK
LBLK = 400
LCH = 40
LSUBS = LBLK // LCH
EPT = E // NS
RPT = 640
RPT_LAST = N - 15 * RPT


def _sc_mesh():
  return plsc.VectorSubcoreMesh(
      core_axis_name="c", subcore_axis_name="s", num_cores=NC,
      num_subcores=NS)


@functools.cache
def _make_agg(with_deg):
  out_type = [jax.ShapeDtypeStruct((2, NC, N, D2), jnp.float32)]
  scratch = [
      pltpu.VMEM((BLKE,), jnp.int32),
      pltpu.VMEM((SUBS, CHUNK), jnp.int32),
      pltpu.VMEM((CHUNK, D2), jnp.float32),
      pltpu.VMEM((CHUNK, D2), jnp.float32),
      pltpu.VMEM((RPT, D2), jnp.float32),
      pltpu.VMEM_SHARED((N, D2), jnp.float32),
      pltpu.SemaphoreType.DMA,
  ]
  if with_deg:
    out_type.append(jax.ShapeDtypeStruct((2, N, L), jnp.float32))
    scratch += [
        pltpu.VMEM((CHUNK, L), jnp.float32),
        pltpu.VMEM((RPT, L), jnp.float32),
        pltpu.VMEM_SHARED((N, L), jnp.float32),
    ]

  @functools.partial(
      pl.kernel, out_type=tuple(out_type), mesh=_sc_mesh(),
      scratch_types=tuple(scratch),
      compiler_params=pltpu.CompilerParams(use_tc_tiling_on_sc=False))
  def agg_kernel(src_cat, dst2d, table_cat, z_rows, *refs):
    if with_deg:
      (z_rows16, ones16, agg_out, deg_out, idx_s, idx_d2, rows0, rows1, wb,
       acc_sh, sem, ones_v, wbd, deg_sh) = refs
    else:
      (agg_out, idx_s, idx_d2, rows0, rows1, wb, acc_sh, sem) = refs
    rowbufs = (rows0, rows1)
    cid = lax.axis_index("c")
    sid = lax.axis_index("s")
    row0 = sid * RPT
    cofs = cid * N

    if with_deg:
      pltpu.sync_copy(ones16, ones_v)

    for p in range(2):
      @pl.when(sid < NS - 1)
      def _():
        pltpu.sync_copy(z_rows, acc_sh.at[pl.ds(row0, RPT)])
        if with_deg:
          @pl.when(cid == p)
          def _():
            pltpu.sync_copy(z_rows16, deg_sh.at[pl.ds(row0, RPT)])

      @pl.when(sid == NS - 1)
      def _():
        pltpu.sync_copy(z_rows.at[pl.ds(0, RPT_LAST)],
                        acc_sh.at[pl.ds(row0, RPT_LAST)])
        if with_deg:
          @pl.when(cid == p)
          def _():
            pltpu.sync_copy(z_rows16.at[pl.ds(0, RPT_LAST)],
                            deg_sh.at[pl.ds(row0, RPT_LAST)])

      plsc.subcore_barrier()

      ebase = p * E + sid * EPT

      erow = p * (E // CHUNK) + sid * (EPT // CHUNK)

      def block_body(b, carry):
        boff = ebase + b * BLKE
        brow = erow + b * SUBS
        d1 = pltpu.async_copy(src_cat.at[pl.ds(boff, BLKE)], idx_s, sem)
        d2 = pltpu.async_copy(dst2d.at[pl.ds(brow, SUBS)], idx_d2, sem)
        d1.wait()
        d2.wait()

        def shift(k, c):
          sl = pl.ds(k * L, L)
          idx_s[sl] = idx_s[sl] + cofs
          return c

        lax.fori_loop(0, BLKE // L, shift, 0)

        descs = [None, None]
        descs[0] = pltpu.async_copy(
            table_cat.at[idx_s.at[pl.ds(0, CHUNK)]], rowbufs[0], sem)
        for k in range(SUBS):
          if k + 1 < SUBS:
            descs[(k + 1) % 2] = pltpu.async_copy(
                table_cat.at[idx_s.at[pl.ds((k + 1) * CHUNK, CHUNK)]],
                rowbufs[(k + 1) % 2], sem)
          descs[k % 2].wait()
          pltpu.sync_copy(rowbufs[k % 2], acc_sh.at[idx_d2.at[k]],
                          add=True)
          if with_deg:
            @pl.when(cid == p)
            def _():
              pltpu.sync_copy(ones_v, deg_sh.at[idx_d2.at[k]], add=True)
        return carry

      lax.fori_loop(0, EPT // BLKE, block_body, 0)
      plsc.subcore_barrier()

      @pl.when(sid < NS - 1)
      def _():
        pltpu.sync_copy(acc_sh.at[pl.ds(row0, RPT)], wb)
        pltpu.sync_copy(wb, agg_out.at[p, cid, pl.ds(row0, RPT)])
        if with_deg:
          @pl.when(cid == p)
          def _():
            pltpu.sync_copy(deg_sh.at[pl.ds(row0, RPT)], wbd)
            pltpu.sync_copy(wbd, deg_out.at[p, pl.ds(row0, RPT)])

      @pl.when(sid == NS - 1)
      def _():
        pltpu.sync_copy(acc_sh.at[pl.ds(row0, RPT_LAST)],
                        wb.at[pl.ds(0, RPT_LAST)])
        pltpu.sync_copy(wb.at[pl.ds(0, RPT_LAST)],
                        agg_out.at[p, cid, pl.ds(row0, RPT_LAST)])
        if with_deg:
          @pl.when(cid == p)
          def _():
            pltpu.sync_copy(deg_sh.at[pl.ds(row0, RPT_LAST)],
                            wbd.at[pl.ds(0, RPT_LAST)])
            pltpu.sync_copy(wbd.at[pl.ds(0, RPT_LAST)],
                            deg_out.at[p, pl.ds(row0, RPT_LAST)])

  return agg_kernel


@functools.cache
def _make_loss():
  @functools.partial(
      pl.kernel,
      out_type=(
          jax.ShapeDtypeStruct((T,), jnp.float32),
          jax.ShapeDtypeStruct((T,), jnp.float32),
          jax.ShapeDtypeStruct((T,), jnp.float32),
          jax.ShapeDtypeStruct((NC, NS, 1, L), jnp.float32),
      ),
      mesh=_sc_mesh(),
      compiler_params=pltpu.CompilerParams(
          needs_layout_passes=False, use_tc_tiling_on_sc=False),
      scratch_types=(
          pltpu.VMEM((N, 6), jnp.float32),
          pltpu.VMEM((LBLK,), jnp.int32),
          pltpu.VMEM((LBLK,), jnp.int32),
          pltpu.VMEM((LBLK,), jnp.int32),
          pltpu.VMEM((LCH, D), jnp.float32),
          pltpu.VMEM((LCH, D), jnp.float32),
          pltpu.VMEM((LCH, D), jnp.float32),
          pltpu.VMEM((LCH, D), jnp.float32),
          pltpu.VMEM((LCH, D), jnp.float32),
          pltpu.VMEM((LCH, D), jnp.float32),
          pltpu.VMEM((9, LBLK), jnp.float32),
          pltpu.VMEM((1, L), jnp.float32),
          pltpu.SemaphoreType.DMA,
      ),
  )
  def loss_kernel(z_hbm, pt_hbm, src_cat, dst_cat, surr_cat,
                  l0_out, l1_out, l2_out, hinge_out,
                  pt_v, idx_i, idx_j, idx_k, zi0, zj0, zk0, zi1, zj1, zk1,
                  lbuf, hbuf, sem):
    cid = lax.axis_index("c")
    sid = lax.axis_index("s")
    bufs = ((zi0, zj0, zk0), (zi1, zj1, zk1))
    pltpu.sync_copy(pt_hbm, pt_v)

    goff0 = cid * E
    goff1 = (4 - 2 * cid) * E
    goff2 = (5 - 2 * cid) * E
    sgn = (1.0 - 2.0 * cid.astype(jnp.float32))

    base = sid * EPT

    def fire(k, which):
      sl = pl.ds(k * LCH, LCH)
      zi, zj, zk = bufs[which]
      return (pltpu.async_copy(z_hbm.at[idx_i.at[sl]], zi, sem),
              pltpu.async_copy(z_hbm.at[idx_j.at[sl]], zj, sem),
              pltpu.async_copy(z_hbm.at[idx_k.at[sl]], zk, sem))

    def block_body(b, hacc):
      boff = base + b * LBLK
      eoff = cid * E + boff
      d1 = pltpu.async_copy(src_cat.at[pl.ds(eoff, LBLK)], idx_i, sem)
      d2 = pltpu.async_copy(dst_cat.at[pl.ds(eoff, LBLK)], idx_j, sem)
      d3 = pltpu.async_copy(surr_cat.at[pl.ds(eoff, LBLK)], idx_k, sem)
      d1.wait()
      d2.wait()
      d3.wait()

      descs = [None, None]
      descs[0] = fire(0, 0)
      for k in range(LSUBS):
        if k + 1 < LSUBS:
          descs[(k + 1) % 2] = fire(k + 1, (k + 1) % 2)
        for d in descs[k % 2]:
          d.wait()
        zi, zj, zk = bufs[k % 2]

        def row_body(r, acc, zi=zi, zj=zj, zk=zk):
          aij = jnp.zeros((L,), jnp.float32)
          aik = jnp.zeros((L,), jnp.float32)
          for c in range(D // L):
            vi = zi[r, pl.ds(c * L, L)]
            vj = zj[r, pl.ds(c * L, L)]
            vk = zk[r, pl.ds(c * L, L)]
            dij = vi - vj
            dik = vi - vk
            aij = aij + dij * dij
            aik = aik + dik * dik
          return acc + jnp.maximum(sgn * jnp.sum(aij - aik), 0.0)

        hacc = lax.fori_loop(0, LCH, row_body, hacc)

      for g in range(LBLK // L):
        gsl = pl.ds(g * L, L)
        iv = idx_i[gsl]
        jv = idx_j[gsl]
        kv = idx_k[gsl]
        for p, (av, bv) in enumerate(((iv, jv), (iv, kv), (jv, kv))):
          for c in range(3):
            la = plsc.load_gather(
                pt_v, [av, jnp.full((L,), c, jnp.int32)])
            lb = plsc.load_gather(
                pt_v, [bv, jnp.full((L,), c + 3, jnp.int32)])
            lbuf[3 * p + c, gsl] = la + lb

      for p, goff in enumerate((goff0, goff1, goff2)):
        for c, out in enumerate((l0_out, l1_out, l2_out)):
          pltpu.sync_copy(lbuf.at[3 * p + c],
                          out.at[pl.ds(goff + boff, LBLK)])
      return hacc

    hsum = lax.fori_loop(0, EPT // LBLK, block_body, 0.0)
    hbuf[0, :] = jnp.full((L,), hsum, jnp.float32)
    pltpu.sync_copy(hbuf, hinge_out.at[cid, sid])

  return loss_kernel


_BLK = 1000


def _halves(ref_lo, ref_hi):
  return jnp.concatenate([ref_lo[0, 0], ref_hi[0, 0]], axis=1)


def _sage_core(x, apl, aph, anl, anh, degp_ref, degn_ref,
               wp_ref, bp_ref, wn_ref, bn_ref):
  invp = 1.0 / (degp_ref[...][0, :, :1] + 1.0)
  invn = 1.0 / (degn_ref[...][0, :, :1] + 1.0)
  meanp = (_halves(apl, aph) + x) * invp
  meann = (_halves(anl, anh) + x) * invn
  hp = jnp.tanh(
      jnp.dot(meanp, wp_ref[...], preferred_element_type=jnp.float32)
      + bp_ref[...])
  hn = jnp.tanh(
      jnp.dot(meann, wn_ref[...], preferred_element_type=jnp.float32)
      + bn_ref[...])
  return hp, hn


def _blkmaps(n_agg):
  specs = []
  for s in range(2):
    for h in range(2):
      specs.append(pl.BlockSpec(
          (1, 1, _BLK, D2),
          functools.partial(lambda s, h, i: (s, h, i, 0), s, h)))
  return specs


def _sage1_body(x_ref, apl, aph, anl, anh, degp_ref, degn_ref,
                wp_ref, bp_ref, wn_ref, bn_ref, out_ref):
  hp, hn = _sage_core(x_ref[...], apl, aph, anl, anh, degp_ref, degn_ref,
                      wp_ref, bp_ref, wn_ref, bn_ref)
  out_ref[0] = hp
  out_ref[1] = hn


def _sage_layer1(x, agg4, deg2, wp, bp, wn, bn):
  full = lambda i: (0, 0)
  return pl.pallas_call(
      _sage1_body,
      grid=(N // _BLK,),
      in_specs=[
          pl.BlockSpec((_BLK, D), lambda i: (i, 0)),
          *_blkmaps(agg4),
          pl.BlockSpec((1, _BLK, L), lambda i: (0, i, 0)),
          pl.BlockSpec((1, _BLK, L), lambda i: (1, i, 0)),
          pl.BlockSpec((D, H), full),
          pl.BlockSpec((1, H), full),
          pl.BlockSpec((D, H), full),
          pl.BlockSpec((1, H), full),
      ],
      out_specs=pl.BlockSpec((2, _BLK, H), lambda i: (0, i, 0)),
      out_shape=jax.ShapeDtypeStruct((2, N, H), jnp.float32),
  )(x, agg4, agg4, agg4, agg4, deg2, deg2, wp, bp, wn, bn)


def _sage2_body(cp_ref, cn_ref, apl, aph, anl, anh, degp_ref, degn_ref,
                wp_ref, bp_ref, wn_ref, bn_ref, w1_ref, w2_ref,
                z_ref, pt_ref):
  x = jnp.concatenate([cp_ref[0], cn_ref[0]], axis=1)
  hp, hn = _sage_core(x, apl, aph, anl, anh, degp_ref, degn_ref,
                      wp_ref, bp_ref, wn_ref, bn_ref)
  z = jnp.concatenate([hp, hn], axis=1)
  z_ref[...] = z
  p1 = jnp.dot(z, w1_ref[...], preferred_element_type=jnp.float32)
  p2 = jnp.dot(z, w2_ref[...], preferred_element_type=jnp.float32)
  pt_ref[...] = jnp.concatenate([p1, p2], axis=1)


def _sage_layer2(c2, agg4, deg2, wp, bp, wn, bn, w1, w2):
  full = lambda i: (0, 0)
  return pl.pallas_call(
      _sage2_body,
      grid=(N // _BLK,),
      in_specs=[
          pl.BlockSpec((1, _BLK, H), lambda i: (0, i, 0)),
          pl.BlockSpec((1, _BLK, H), lambda i: (1, i, 0)),
          *_blkmaps(agg4),
          pl.BlockSpec((1, _BLK, L), lambda i: (0, i, 0)),
          pl.BlockSpec((1, _BLK, L), lambda i: (1, i, 0)),
          pl.BlockSpec((D, H), full),
          pl.BlockSpec((1, H), full),
          pl.BlockSpec((D, H), full),
          pl.BlockSpec((1, H), full),
          pl.BlockSpec((D, 3), full),
          pl.BlockSpec((D, 3), full),
      ],
      out_specs=[
          pl.BlockSpec((_BLK, D), lambda i: (i, 0)),
          pl.BlockSpec((_BLK, 6), lambda i: (i, 0)),
      ],
      out_shape=[
          jax.ShapeDtypeStruct((N, D), jnp.float32),
          jax.ShapeDtypeStruct((N, 6), jnp.float32),
      ],
  )(c2, c2, agg4, agg4, agg4, agg4, deg2, deg2, wp, bp, wn, bn, w1, w2)


_TROWS = T // D


def _final_body(l0_ref, l1_ref, l2_ref, tgt_ref, hinge_ref,
                wp0_ref, wn0_ref, wp1_ref, wn1_ref, rw_ref, out_ref):
  i = pl.program_id(0)
  l0 = l0_ref[...]
  l1 = l1_ref[...]
  l2 = l2_ref[...]
  t = tgt_ref[...]
  m = jnp.maximum(jnp.maximum(l0, l1), l2)
  lse = m + jnp.log(jnp.exp(l0 - m) + jnp.exp(l1 - m) + jnp.exp(l2 - m))
  lt = jnp.where(t == 0, l0, jnp.where(t == 1, l1, l2))
  part = jnp.sum(lse - lt)

  @pl.when(i == 0)
  def _():
    out_ref[...] = jnp.zeros((1, 1), jnp.float32)

  out_ref[...] += jnp.full((1, 1), part / T, jnp.float32)

  @pl.when(i == pl.num_programs(0) - 1)
  def _():
    hp = jnp.sum(hinge_ref[0, :, 0]) / E
    hn = jnp.sum(hinge_ref[1, :, 0]) / E

    def rnm(w):
      return jnp.mean(jnp.sqrt(jnp.sum(w * w, axis=1)))

    reg = (rnm(wp0_ref[...]) + rnm(wn0_ref[...]) + rnm(rw_ref[...])
           + rnm(wp1_ref[...]) + rnm(wn1_ref[...]))
    out_ref[...] += jnp.full((1, 1), hp + hn + 0.01 * reg, jnp.float32)


def _final_loss(l0, l1, l2, tgt, hinge, wp0, wn0, wp1, wn1, rw):
  full = lambda i: (0, 0)
  full3 = lambda i: (0, 0, 0)
  return pl.pallas_call(
      _final_body,
      grid=(1,),
      in_specs=[
          pl.BlockSpec((_TROWS, D), lambda i: (i, 0)),
          pl.BlockSpec((_TROWS, D), lambda i: (i, 0)),
          pl.BlockSpec((_TROWS, D), lambda i: (i, 0)),
          pl.BlockSpec((_TROWS, D), lambda i: (i, 0)),
          pl.BlockSpec((NC, NS, L), full3),
          pl.BlockSpec((D, H), full),
          pl.BlockSpec((D, H), full),
          pl.BlockSpec((D, H), full),
          pl.BlockSpec((D, H), full),
          pl.BlockSpec((2 * D, 3), full),
      ],
      out_specs=pl.BlockSpec((1, 1), full),
      out_shape=jax.ShapeDtypeStruct((1, 1), jnp.float32),
  )(l0, l1, l2, tgt, hinge, wp0, wn0, wp1, wn1, rw)


def kernel(positive_edges, negative_edges, target, pos_surr, neg_surr, X,
           W_pos0, b_pos0, W_neg0, b_neg0, W_pos1, b_pos1, W_neg1, b_neg1,
           reg_W):
  src_cat = jnp.concatenate([positive_edges[0], negative_edges[0]])
  dst_cat = jnp.concatenate([positive_edges[1], negative_edges[1]])
  surr_cat = jnp.concatenate([pos_surr, neg_surr])
  x_cat = jnp.concatenate([X[:, :D2], X[:, D2:]], axis=0)
  z_rows = jnp.zeros((RPT, D2), jnp.float32)
  z_rows16 = jnp.zeros((RPT, L), jnp.float32)
  ones16 = jnp.ones((CHUNK, L), jnp.float32)

  dst2d = dst_cat.reshape(-1, CHUNK)
  agg4a, deg2 = _make_agg(True)(src_cat, dst2d, x_cat, z_rows, z_rows16,
                                ones16)
  c2 = _sage_layer1(X, agg4a, deg2,
                    W_pos0, b_pos0[None, :], W_neg0, b_neg0[None, :])
  agg4b = _make_agg(False)(src_cat, dst2d, c2.reshape(2 * N, H), z_rows)
  if isinstance(agg4b, (tuple, list)):
    agg4b = agg4b[0]
  w_neg1_sw = jnp.concatenate([W_neg1[H:], W_neg1[:H]], axis=0)
  z, pt = _sage_layer2(c2, agg4b, deg2,
                       W_pos1, b_pos1[None, :], w_neg1_sw, b_neg1[None, :],
                       reg_W[:D], reg_W[D:])
  l0, l1, l2, hinge = _make_loss()(z, pt, src_cat, dst_cat, surr_cat)
  hinge = hinge.reshape(NC, NS, L)
  loss = _final_loss(l0.reshape(_TROWS, D), l1.reshape(_TROWS, D),
                     l2.reshape(_TROWS, D), target.reshape(_TROWS, D),
                     hinge, W_pos0, W_neg0, W_pos1, W_neg1, reg_W)
  return (loss[0, 0], z)
